# baseline (speedup 1.0000x reference)
import functools

import jax
import jax.numpy as jnp
from jax import lax
from jax.experimental import pallas as pl
from jax.experimental.pallas import tpu as pltpu
from jax.experimental.pallas import tpu_sc as plsc

DIM = 64
SCALE = 8.0
LANES = 16

NC = 2
NS = 16
NW = NC * NS

NBATCH = 4096
SEQ = 200
PER_W = NBATCH // NW
S0 = 128
S1 = SEQ - S0
NBUF = 4
ROWS_PER_ITER = 4


def _scale_buf(buf):
    def body(i, c):
        for r in range(ROWS_PER_ITER):
            for j in range(DIM // LANES):
                sl = (i * ROWS_PER_ITER + r, pl.ds(j * LANES, LANES))
                buf[sl] = buf[sl] * SCALE
        return c

    lax.fori_loop(0, SEQ // ROWS_PER_ITER, body, 0)


def _emb_body(x_hbm, lut_hbm, out_hbm, idx_v, rows, isem, gsem, ssem):
    wid = lax.axis_index("s") * NC + lax.axis_index("c")
    b0 = wid * PER_W

    def stage(t, c):
        for s in range(4):
            lb = t * 4 + s
            b = b0 + lb
            pltpu.async_copy(x_hbm.at[b, pl.ds(0, S0)], idx_v.at[lb, 0], isem.at[s])
            pltpu.async_copy(
                x_hbm.at[b, pl.ds(S0, S1)], idx_v.at[lb, 1, pl.ds(0, S1)], isem.at[s]
            )
        return c

    lax.fori_loop(0, PER_W // 4, stage, 0)

    def drain(t, c):
        for s in range(4):
            lb = t * 4 + s
            b = b0 + lb
            pltpu.make_async_copy(
                x_hbm.at[b, pl.ds(0, S0)], idx_v.at[lb, 0], isem.at[s]
            ).wait()
            pltpu.make_async_copy(
                x_hbm.at[b, pl.ds(S0, S1)], idx_v.at[lb, 1, pl.ds(0, S1)], isem.at[s]
            ).wait()
        return c

    lax.fori_loop(0, PER_W // 4, drain, 0)

    def start_gather(s, lb):
        pltpu.async_copy(
            lut_hbm.at[idx_v.at[lb, 0]], rows.at[s, pl.ds(0, S0)], gsem.at[s]
        )
        pltpu.async_copy(
            lut_hbm.at[idx_v.at[lb, 1, pl.ds(0, S1)]],
            rows.at[s, pl.ds(S0, S1)],
            gsem.at[s],
        )

    def wait_gather(s, lb):
        pltpu.make_async_copy(
            lut_hbm.at[idx_v.at[lb, 0]], rows.at[s, pl.ds(0, S0)], gsem.at[s]
        ).wait()
        pltpu.make_async_copy(
            lut_hbm.at[idx_v.at[lb, 1, pl.ds(0, S1)]],
            rows.at[s, pl.ds(S0, S1)],
            gsem.at[s],
        ).wait()

    def start_store(s, lb):
        pltpu.async_copy(rows.at[s], out_hbm.at[b0 + lb], ssem.at[s])

    def wait_store(s, lb):
        pltpu.make_async_copy(rows.at[s], out_hbm.at[b0 + lb], ssem.at[s]).wait()

    for s in range(NBUF):
        start_gather(s, s)

    def group(t, carry):
        g0 = t * NBUF
        for s in range(NBUF):
            lb = g0 + s
            wait_gather(s, lb)
            _scale_buf(rows.at[s])
            start_store(s, lb)
        for s in range(NBUF):
            lb = g0 + s
            wait_store(s, lb)
            start_gather(s, lb + NBUF)
        return carry

    lax.fori_loop(0, PER_W // NBUF - 1, group, 0)

    g0 = PER_W - NBUF
    for s in range(NBUF):
        lb = g0 + s
        wait_gather(s, lb)
        _scale_buf(rows.at[s])
        start_store(s, lb)
    for s in range(NBUF):
        wait_store(s, g0 + s)


@functools.partial(
    pl.kernel,
    out_type=jax.ShapeDtypeStruct((NBATCH, SEQ, DIM), jnp.float32),
    mesh=plsc.VectorSubcoreMesh(core_axis_name="c", subcore_axis_name="s"),
    compiler_params=pltpu.CompilerParams(use_tc_tiling_on_sc=False),
    scratch_types=[
        pltpu.VMEM((PER_W, 2, 128), jnp.int32),
        pltpu.VMEM((NBUF, SEQ, DIM), jnp.float32),
        pltpu.SemaphoreType.DMA((4,)),
        pltpu.SemaphoreType.DMA((NBUF,)),
        pltpu.SemaphoreType.DMA((NBUF,)),
    ],
)
def _emb(x_hbm, lut_hbm, out_hbm, idx_v, rows, isem, gsem, ssem):
    _emb_body(x_hbm, lut_hbm, out_hbm, idx_v, rows, isem, gsem, ssem)


def kernel(x, lut):
    return _emb(x.astype(jnp.int32), lut)


# speedup vs baseline: 1.0247x; 1.0247x over previous
import functools

import jax
import jax.numpy as jnp
from jax import lax
from jax.experimental import pallas as pl
from jax.experimental.pallas import tpu as pltpu
from jax.experimental.pallas import tpu_sc as plsc

DIM = 64
SCALE = 8.0
LANES = 16

NC = 2
NS = 16
NW = NC * NS

VOCAB = 1000000
NBATCH = 4096
SEQ = 200
PER_W = NBATCH // NW
S0 = 128
S1 = SEQ - S0
NBUF = 4
ROWS_PER_ITER = 4

DCHUNK = 200
NDCHUNK = VOCAB // DCHUNK
CPT = -(-NDCHUNK // NW)
NA2 = -(-CPT // 2)


def _scale_buf(buf):
    def body(i, c):
        for r in range(ROWS_PER_ITER):
            for j in range(DIM // LANES):
                sl = (i * ROWS_PER_ITER + r, pl.ds(j * LANES, LANES))
                buf[sl] = buf[sl] * SCALE
        return c

    lax.fori_loop(0, SEQ // ROWS_PER_ITER, body, 0)


def _emb_body(x_hbm, lut_hbm, out_hbm, dense_hbm, idx_v, rows, isem, gsem, ssem, xsem):
    cid_ax = lax.axis_index("c")
    sid = lax.axis_index("s")
    wid = sid * NC + cid_ax

    def a_read(b, i):
        base = (i * NW + wid) * DCHUNK
        pltpu.async_copy(
            lut_hbm.at[pl.ds(base, DCHUNK)], rows.at[b], gsem.at[b]
        )

    def a_wait_read(b, i):
        base = (i * NW + wid) * DCHUNK
        pltpu.make_async_copy(
            lut_hbm.at[pl.ds(base, DCHUNK)], rows.at[b], gsem.at[b]
        ).wait()

    def a_valid(i):
        return i * NW + wid < NDCHUNK

    @pl.when(a_valid(0))
    def _():
        a_read(0, 0)

    def a_group(t, carry):
        for s in range(2):
            i = t * 2 + s

            @pl.when(a_valid(i + 1))
            def _():
                a_read((s + 1) % 2, i + 1)

            @pl.when(a_valid(i))
            def _():
                a_wait_read(s, i)
                base = (i * NW + wid) * DCHUNK
                pltpu.sync_copy(rows.at[s], dense_hbm.at[pl.ds(base, DCHUNK)])

        return carry

    lax.fori_loop(0, NA2, a_group, 0)

    plsc.subcore_barrier()

    @pl.when(sid == 0)
    def _():
        pl.semaphore_signal(xsem, 1, core_index=1 - cid_ax)
        pl.semaphore_wait(xsem, 1)

    plsc.subcore_barrier()

    b0 = wid * PER_W

    def stage_idx(s, lb):
        pltpu.async_copy(x_hbm.at[wid, lb], idx_v.at[s], isem.at[s])

    def wait_idx(s, lb):
        pltpu.make_async_copy(x_hbm.at[wid, lb], idx_v.at[s], isem.at[s]).wait()

    def start_gather(s, lb):
        pltpu.async_copy(
            dense_hbm.at[idx_v.at[s, 0]], rows.at[s, pl.ds(0, S0)], gsem.at[s]
        )
        pltpu.async_copy(
            dense_hbm.at[idx_v.at[s, 1, pl.ds(0, S1)]],
            rows.at[s, pl.ds(S0, S1)],
            gsem.at[s],
        )

    def wait_gather(s, lb):
        pltpu.make_async_copy(
            dense_hbm.at[idx_v.at[s, 0]], rows.at[s, pl.ds(0, S0)], gsem.at[s]
        ).wait()
        pltpu.make_async_copy(
            dense_hbm.at[idx_v.at[s, 1, pl.ds(0, S1)]],
            rows.at[s, pl.ds(S0, S1)],
            gsem.at[s],
        ).wait()

    def start_store(s, lb):
        pltpu.async_copy(rows.at[s], out_hbm.at[b0 + lb], ssem.at[s])

    def wait_store(s, lb):
        pltpu.make_async_copy(rows.at[s], out_hbm.at[b0 + lb], ssem.at[s]).wait()

    for s in range(NBUF):
        stage_idx(s, s)
    for s in range(NBUF):
        wait_idx(s, s)
        start_gather(s, s)

    def group(t, carry):
        g0 = t * NBUF
        for s in range(NBUF):
            lb = g0 + s
            wait_gather(s, lb)
            _scale_buf(rows.at[s])
            start_store(s, lb)
            stage_idx(s, lb + NBUF)
        for s in range(NBUF):
            lb = g0 + s
            wait_store(s, lb)
            wait_idx(s, lb + NBUF)
            start_gather(s, lb + NBUF)
        return carry

    lax.fori_loop(0, PER_W // NBUF - 1, group, 0)

    g0 = PER_W - NBUF
    for s in range(NBUF):
        lb = g0 + s
        wait_gather(s, lb)
        _scale_buf(rows.at[s])
        start_store(s, lb)
    for s in range(NBUF):
        wait_store(s, g0 + s)


@functools.partial(
    pl.kernel,
    out_type=jax.ShapeDtypeStruct((NBATCH, SEQ, DIM), jnp.float32),
    mesh=plsc.VectorSubcoreMesh(core_axis_name="c", subcore_axis_name="s"),
    scratch_types=[
        pltpu.HBM((VOCAB, DIM), jnp.float32),
        pltpu.VMEM((NBUF, 2, 128), jnp.int32),
        pltpu.VMEM((NBUF, DCHUNK, DIM), jnp.float32),
        pltpu.SemaphoreType.DMA((NBUF,)),
        pltpu.SemaphoreType.DMA((NBUF,)),
        pltpu.SemaphoreType.DMA((NBUF,)),
        pltpu.SemaphoreType.REGULAR,
    ],
)
def _emb(x_hbm, lut_hbm, out_hbm, dense_hbm, idx_v, rows, isem, gsem, ssem, xsem):
    _emb_body(x_hbm, lut_hbm, out_hbm, dense_hbm, idx_v, rows, isem, gsem, ssem, xsem)


def kernel(x, lut):
    xp = jnp.pad(x.astype(jnp.int32), ((0, 0), (0, 256 - SEQ)))
    x4 = xp.reshape(NW, PER_W, 2, 128)
    return _emb(x4, lut)
